# Initial kernel scaffold; baseline (speedup 1.0000x reference)
import jax
import jax.numpy as jnp
from jax.experimental import pallas as pl


def kernel(input_ids, weight):
    raise NotImplementedError("write your pallas kernel here")


# speedup vs baseline: 1.1032x; 1.1032x over previous
import functools

import jax
import jax.numpy as jnp
from jax import lax
from jax.experimental import pallas as pl
from jax.experimental.pallas import tpu as pltpu
from jax.experimental.pallas import tpu_sc as plsc

EMB_DIM = 32
SCALE = float(EMB_DIM ** 0.5)

NUM_CORES = 2
NUM_SUBCORES = 16
NUM_WORKERS = NUM_CORES * NUM_SUBCORES
LANES = 16

CHUNK = 128


@functools.partial(jax.jit, static_argnames=("total",))
def _embed_flat(idx_flat, weight, *, total):
    n_per_w = total // NUM_WORKERS
    n_chunks = n_per_w // CHUNK
    mesh = plsc.VectorSubcoreMesh(core_axis_name="c", subcore_axis_name="s")

    @functools.partial(
        pl.kernel,
        mesh=mesh,
        compiler_params=pltpu.CompilerParams(use_tc_tiling_on_sc=False),
        out_type=jax.ShapeDtypeStruct((total, EMB_DIM), jnp.float32),
        scratch_types=[
            pltpu.VMEM((CHUNK,), jnp.int32),
            pltpu.VMEM((CHUNK, EMB_DIM), jnp.float32),
            pltpu.SemaphoreType.DMA,
        ],
    )
    def k(idx_hbm, table_hbm, out_hbm, idx_v, rows_v, sem):
        wid = lax.axis_index("s") * NUM_CORES + lax.axis_index("c")
        base = wid * n_per_w

        def chunk_body(g, carry):
            off = base + g * CHUNK
            pltpu.sync_copy(idx_hbm.at[pl.ds(off, CHUNK)], idx_v)
            pltpu.async_copy(table_hbm.at[idx_v], rows_v, sem).wait()

            def scale_body(i, c):
                rows_v[i, pl.ds(0, LANES)] = rows_v[i, pl.ds(0, LANES)] * SCALE
                rows_v[i, pl.ds(LANES, LANES)] = (
                    rows_v[i, pl.ds(LANES, LANES)] * SCALE
                )
                return c

            lax.fori_loop(0, CHUNK, scale_body, 0)
            pltpu.sync_copy(rows_v, out_hbm.at[pl.ds(off, CHUNK)])
            return carry

        lax.fori_loop(0, n_chunks, chunk_body, 0)

    return k(idx_flat, weight)


def kernel(input_ids, weight):
    b, s = input_ids.shape
    total = b * s
    idx_flat = input_ids.reshape(total).astype(jnp.int32)
    out = _embed_flat(idx_flat, weight, total=total)
    return out.reshape(b, s, EMB_DIM)


# speedup vs baseline: 1.4653x; 1.3282x over previous
import functools

import jax
import jax.numpy as jnp
from jax import lax
from jax.experimental import pallas as pl
from jax.experimental.pallas import tpu as pltpu
from jax.experimental.pallas import tpu_sc as plsc

EMB_DIM = 32
SCALE = float(EMB_DIM ** 0.5)

NUM_CORES = 2
NUM_SUBCORES = 16
NUM_WORKERS = NUM_CORES * NUM_SUBCORES
LANES = 16

IDX_PER_STREAM = 128
STREAMS_PER_CHUNK = 10
CHUNK = IDX_PER_STREAM * STREAMS_PER_CHUNK
UNROLL = 8


@functools.partial(jax.jit, static_argnames=("total",))
def _embed_flat(idx_flat, weight, *, total):
    n_per_w = total // NUM_WORKERS
    n_chunks = n_per_w // CHUNK
    n_pairs = n_chunks // 2
    assert n_pairs * 2 * CHUNK == n_per_w
    mesh = plsc.VectorSubcoreMesh(core_axis_name="c", subcore_axis_name="s")

    @functools.partial(
        pl.kernel,
        mesh=mesh,
        compiler_params=pltpu.CompilerParams(use_tc_tiling_on_sc=False),
        out_type=jax.ShapeDtypeStruct((total, EMB_DIM), jnp.float32),
        scratch_types=[
            pltpu.VMEM((CHUNK,), jnp.int32),
            pltpu.VMEM((CHUNK,), jnp.int32),
            pltpu.VMEM((CHUNK, EMB_DIM), jnp.float32),
            pltpu.VMEM((CHUNK, EMB_DIM), jnp.float32),
            pltpu.SemaphoreType.DMA,
            pltpu.SemaphoreType.DMA,
            pltpu.SemaphoreType.DMA,
            pltpu.SemaphoreType.DMA,
        ],
    )
    def k(idx_hbm, table_hbm, out_hbm, idx_a, idx_b, rows_a, rows_b,
          gsem_a, gsem_b, wsem_a, wsem_b):
        wid = lax.axis_index("s") * NUM_CORES + lax.axis_index("c")
        base = wid * n_per_w

        def fire_gathers(idx_v, rows_v, sem):
            for t in range(STREAMS_PER_CHUNK):
                sl = pl.ds(t * IDX_PER_STREAM, IDX_PER_STREAM)
                pltpu.async_copy(table_hbm.at[idx_v.at[sl]], rows_v.at[sl], sem)

        def drain_gathers(idx_v, rows_v, sem):
            for t in range(STREAMS_PER_CHUNK):
                sl = pl.ds(t * IDX_PER_STREAM, IDX_PER_STREAM)
                pltpu.make_async_copy(
                    table_hbm.at[idx_v.at[sl]], rows_v.at[sl], sem).wait()

        def scale_rows(rows_v):
            def body(i, c):
                for u in range(UNROLL):
                    r = i * UNROLL + u
                    rows_v[r, pl.ds(0, LANES)] = rows_v[r, pl.ds(0, LANES)] * SCALE
                    rows_v[r, pl.ds(LANES, LANES)] = (
                        rows_v[r, pl.ds(LANES, LANES)] * SCALE)
                return c
            lax.fori_loop(0, CHUNK // UNROLL, body, 0)

        pltpu.sync_copy(idx_hbm.at[pl.ds(base, CHUNK)], idx_a)
        fire_gathers(idx_a, rows_a, gsem_a)

        def pair_body(j, carry):
            off_a = base + (2 * j) * CHUNK
            off_b = off_a + CHUNK

            @pl.when(j > 0)
            def _():
                pltpu.make_async_copy(rows_b, out_hbm.at[pl.ds(off_b, CHUNK)],
                                      wsem_b).wait()

            pltpu.sync_copy(idx_hbm.at[pl.ds(off_b, CHUNK)], idx_b)
            fire_gathers(idx_b, rows_b, gsem_b)

            drain_gathers(idx_a, rows_a, gsem_a)
            scale_rows(rows_a)
            pltpu.async_copy(rows_a, out_hbm.at[pl.ds(off_a, CHUNK)], wsem_a)

            drain_gathers(idx_b, rows_b, gsem_b)
            scale_rows(rows_b)

            pltpu.make_async_copy(rows_a, out_hbm.at[pl.ds(off_a, CHUNK)],
                                  wsem_a).wait()

            @pl.when(j < n_pairs - 1)
            def _():
                off_n = off_a + 2 * CHUNK
                pltpu.sync_copy(idx_hbm.at[pl.ds(off_n, CHUNK)], idx_a)
                fire_gathers(idx_a, rows_a, gsem_a)

            pltpu.async_copy(rows_b, out_hbm.at[pl.ds(off_b, CHUNK)], wsem_b)
            return carry

        lax.fori_loop(0, n_pairs, pair_body, 0)

        last_off = base + n_per_w - CHUNK
        pltpu.make_async_copy(rows_b, out_hbm.at[pl.ds(last_off, CHUNK)],
                              wsem_b).wait()

    return k(idx_flat, weight)


def kernel(input_ids, weight):
    b, s = input_ids.shape
    total = b * s
    idx_flat = input_ids.reshape(total).astype(jnp.int32)
    out = _embed_flat(idx_flat, weight, total=total)
    return out.reshape(b, s, EMB_DIM)
